# baseline (speedup 1.0000x reference)
import jax
import jax.numpy as jnp
from jax import lax
from jax.experimental import pallas as pl
from jax.experimental.pallas import tpu as pltpu
from jax.experimental.pallas import tpu_sc as plsc

BATCH = 16384
DIM = 64
NC = 2
NS = 16
NW = NC * NS
BPW = BATCH // NW
PASS = 128
NPASS = BPW // PASS
PGROUPS = PASS // 16


def _sc1_body(uidx_hbm, iidx_hbm, gm_hbm, umean_hbm, imean_hbm, iimp_hbm,
              part_hbm, irowsout_hbm,
              uidx_v, iidx_v, irows_v, ubias_v, ibias_v, gm_v, part_v,
              sem, bsem):
    wid = lax.axis_index("s") * NC + lax.axis_index("c")
    base = wid * BPW

    pltpu.sync_copy(uidx_hbm.at[wid], uidx_v)
    pltpu.sync_copy(iidx_hbm.at[wid], iidx_v)
    pltpu.sync_copy(gm_hbm, gm_v)

    bias_copies = []
    for c in range(NPASS):
        dst = pl.ds(c * PASS, PASS)
        bias_copies.append(pltpu.async_copy(
            umean_hbm.at[uidx_v.at[c]], ubias_v.at[dst], bsem))
        bias_copies.append(pltpu.async_copy(
            imean_hbm.at[iidx_v.at[c]], ibias_v.at[dst], bsem))

    def fetch(jv, carry):
        j16 = jv * 16
        p = jv // (PASS // 16)
        k16 = j16 - p * PASS
        ivec = iidx_v[p, pl.ds(k16, 16)]
        for k in range(16):
            pltpu.async_copy(
                iimp_hbm.at[ivec[k]], irows_v.at[j16 + k], sem)
        return carry

    lax.fori_loop(0, BPW // 16, fetch, 0)

    for cp in bias_copies:
        cp.wait()
    gmv = gm_v[...]
    for g in range(BPW // 16):
        sl = pl.ds(g * 16, 16)
        part_v[sl] = gmv + ubias_v[sl] + ibias_v[sl]

    pltpu.make_async_copy(
        iimp_hbm.at[pl.ds(0, BPW)], irows_v, sem).wait()
    pltpu.sync_copy(part_v, part_hbm.at[pl.ds(base, BPW)])
    pltpu.sync_copy(irows_v, irowsout_hbm.at[pl.ds(base, BPW)])


def _sc2_body(uidx_hbm, part_hbm, irows_hbm, uimp_hbm, out_hbm,
              uidx_v, urows_v, irows_v, part_v, out_v, sems, isem):
    wid = lax.axis_index("s") * NC + lax.axis_index("c")
    base = wid * BPW

    pltpu.sync_copy(uidx_hbm.at[wid], uidx_v)
    icopy = pltpu.async_copy(
        irows_hbm.at[pl.ds(base, BPW)], irows_v, isem)
    pcopy = pltpu.async_copy(
        part_hbm.at[pl.ds(base, BPW)], part_v, isem)

    def fire(p):
        buf = p & 1
        sem = sems.at[buf]

        def fetch(jv, carry):
            j16 = jv * 16
            uvec = uidx_v[p, pl.ds(j16, 16)]
            for k in range(16):
                pltpu.async_copy(
                    uimp_hbm.at[uvec[k]], urows_v.at[buf, j16 + k], sem)
            return carry

        lax.fori_loop(0, PASS // 16, fetch, 0)

    def drain(p):
        buf = p & 1
        pltpu.make_async_copy(
            uimp_hbm.at[pl.ds(0, PASS)], urows_v.at[buf], sems.at[buf]).wait()

    lane = jnp.arange(16, dtype=jnp.int32)

    def compute(p):
        buf = p & 1
        urows = urows_v.at[buf]
        for g in range(PGROUPS):
            row0 = g * 16
            rows = row0 + lane
            b0 = p * PASS + row0
            sl = pl.ds(b0, 16)
            brows = b0 + lane
            acc = part_v[sl]
            for d in range(DIM):
                cols = jnp.full((16,), d, dtype=jnp.int32)
                u = plsc.load_gather(urows, [rows, cols])
                v = plsc.load_gather(irows_v, [brows, cols])
                acc = acc + u * v
            out_v[sl] = acc

    fire(0)
    icopy.wait()
    pcopy.wait()
    for p in range(NPASS):
        if p + 1 < NPASS:
            fire(p + 1)
        drain(p)
        compute(p)

    pltpu.sync_copy(out_v, out_hbm.at[pl.ds(base, BPW)])


def kernel(user_mapped, item_mapped, global_mean, user_mean, item_mean,
           user_implicit, item_implicit):
    uidx3 = user_mapped.reshape(NW, NPASS, PASS)
    iidx3 = item_mapped.reshape(NW, NPASS, PASS)
    gm_vec = jnp.broadcast_to(global_mean.astype(jnp.float32), (16,))

    mesh = plsc.VectorSubcoreMesh(
        core_axis_name="c", subcore_axis_name="s",
        num_cores=NC, num_subcores=NS)
    cparams = pltpu.CompilerParams(
        needs_layout_passes=False, use_tc_tiling_on_sc=True)

    run1 = pl.kernel(
        _sc1_body,
        out_type=(jax.ShapeDtypeStruct((BATCH,), jnp.float32),
                  jax.ShapeDtypeStruct((BATCH, DIM), jnp.float32)),
        mesh=mesh,
        compiler_params=cparams,
        scratch_types=[
            pltpu.VMEM((NPASS, PASS), jnp.int32),
            pltpu.VMEM((NPASS, PASS), jnp.int32),
            pltpu.VMEM((BPW, DIM), jnp.float32),
            pltpu.VMEM((BPW,), jnp.float32),
            pltpu.VMEM((BPW,), jnp.float32),
            pltpu.VMEM((16,), jnp.float32),
            pltpu.VMEM((BPW,), jnp.float32),
            pltpu.SemaphoreType.DMA,
            pltpu.SemaphoreType.DMA,
        ],
    )
    part, irows = run1(uidx3, iidx3, gm_vec, user_mean, item_mean,
                       item_implicit)

    run2 = pl.kernel(
        _sc2_body,
        out_type=jax.ShapeDtypeStruct((BATCH,), jnp.float32),
        mesh=mesh,
        compiler_params=cparams,
        scratch_types=[
            pltpu.VMEM((NPASS, PASS), jnp.int32),
            pltpu.VMEM((2, PASS, DIM), jnp.float32),
            pltpu.VMEM((BPW, DIM), jnp.float32),
            pltpu.VMEM((BPW,), jnp.float32),
            pltpu.VMEM((BPW,), jnp.float32),
            pltpu.SemaphoreType.DMA((2,)),
            pltpu.SemaphoreType.DMA,
        ],
    )
    return run2(uidx3, part, irows, user_implicit)
